# baseline (speedup 1.0000x reference)
import functools
import math

import jax
import jax.numpy as jnp
from jax import lax
from jax.experimental import pallas as pl
from jax.experimental.pallas import tpu as pltpu
from jax.experimental.pallas import tpu_sc as plsc

D_MODEL = 64
CHUNK = 128
LANES = 16
NBUF = 4


@functools.cache
def _build(B: int, V: int, D: int):
    info = plsc.get_sparse_core_info()
    nc, ns = info.num_cores, info.num_subcores
    nw = nc * ns
    b_per_w = B // nw
    n_chunks = b_per_w // CHUNK
    scale = math.sqrt(D)

    mesh = plsc.VectorSubcoreMesh(core_axis_name="c", subcore_axis_name="s")

    @functools.partial(
        pl.kernel,
        out_type=jax.ShapeDtypeStruct((B, D), jnp.float32),
        mesh=mesh,
        scratch_types=(
            [pltpu.VMEM((n_chunks, CHUNK), jnp.int32)]
            + [pltpu.VMEM((CHUNK, D), jnp.float32) for _ in range(2 * NBUF)]
            + [pltpu.SemaphoreType.DMA for _ in range(2 * NBUF)]
        ),
        compiler_params=pltpu.CompilerParams(use_tc_tiling_on_sc=False),
    )
    def emb(idx_hbm, tbl_hbm, out_hbm, *scratch):
        idx_v = scratch[0]
        gbuf = scratch[1 : 1 + NBUF]
        obuf = scratch[1 + NBUF : 1 + 2 * NBUF]
        gsem = scratch[1 + 2 * NBUF : 1 + 3 * NBUF]
        osem = scratch[1 + 3 * NBUF : 1 + 4 * NBUF]

        wid = lax.axis_index("s") * nc + lax.axis_index("c")
        chunk0 = wid * n_chunks
        pltpu.sync_copy(idx_hbm.at[pl.ds(chunk0, n_chunks)], idx_v)

        for b in range(NBUF):
            pltpu.async_copy(tbl_hbm.at[idx_v.at[b]], gbuf[b], gsem[b])

        def outer(i, carry):
            og = i * NBUF
            for b in range(NBUF):
                jg = og + b
                @pl.when(jg >= NBUF)
                def _():
                    pltpu.make_async_copy(
                        obuf[b], out_hbm.at[pl.ds(chunk0 * CHUNK, CHUNK)],
                        osem[b],
                    ).wait()

                pltpu.make_async_copy(
                    tbl_hbm.at[idx_v.at[jg]], gbuf[b], gsem[b]
                ).wait()

                def scale_row(r, c2, _gb=gbuf[b], _ob=obuf[b]):
                    for c in range(D // LANES):
                        sl = pl.ds(c * LANES, LANES)
                        _ob[r, sl] = _gb[r, sl] * scale
                    return c2

                lax.fori_loop(0, CHUNK, scale_row, 0, unroll=4)

                @pl.when(jg + NBUF < n_chunks)
                def _():
                    pltpu.async_copy(
                        tbl_hbm.at[idx_v.at[jg + NBUF]], gbuf[b], gsem[b]
                    )

                pltpu.async_copy(
                    obuf[b],
                    out_hbm.at[pl.ds((chunk0 + jg) * CHUNK, CHUNK)],
                    osem[b],
                )
            return carry

        lax.fori_loop(0, n_chunks // NBUF, outer, 0)

        for b in range(NBUF):
            pltpu.make_async_copy(
                obuf[b], out_hbm.at[pl.ds(chunk0 * CHUNK, CHUNK)], osem[b]
            ).wait()

    return emb


def kernel(x, table):
    b0, b1 = x.shape
    B = b0 * b1
    V, D = table.shape
    idx = x.reshape(B // CHUNK, CHUNK).astype(jnp.int32)
    out = _build(B, V, D)(idx, table)
    return out.reshape(b0, b1, D)


# speedup vs baseline: 1.2637x; 1.2637x over previous
import functools
import math

import jax
import jax.numpy as jnp
from jax import lax
from jax.experimental import pallas as pl
from jax.experimental.pallas import tpu as pltpu
from jax.experimental.pallas import tpu_sc as plsc

D_MODEL = 64
CHUNK = 128
LANES = 16
NBUF = 4


@functools.cache
def _build(B: int, V: int, D: int):
    info = plsc.get_sparse_core_info()
    nc, ns = info.num_cores, info.num_subcores
    nw = nc * ns
    b_per_w = B // nw
    n_chunks = b_per_w // CHUNK
    scale = math.sqrt(D)
    out_rows_per_chunk = CHUNK * D // 128

    mesh = plsc.VectorSubcoreMesh(core_axis_name="c", subcore_axis_name="s")

    @functools.partial(
        pl.kernel,
        out_type=jax.ShapeDtypeStruct((B * D // 128, 128), jnp.float32),
        mesh=mesh,
        scratch_types=(
            [pltpu.VMEM((n_chunks, CHUNK), jnp.int32)]
            + [pltpu.VMEM((CHUNK, D), jnp.float32) for _ in range(NBUF)]
            + [
                pltpu.VMEM((out_rows_per_chunk, 128), jnp.float32)
                for _ in range(NBUF)
            ]
            + [pltpu.SemaphoreType.DMA for _ in range(2 * NBUF)]
        ),
        compiler_params=pltpu.CompilerParams(use_tc_tiling_on_sc=False),
    )
    def emb(idx_hbm, tbl_hbm, out_hbm, *scratch):
        idx_v = scratch[0]
        gbuf = scratch[1 : 1 + NBUF]
        obuf = scratch[1 + NBUF : 1 + 2 * NBUF]
        gsem = scratch[1 + 2 * NBUF : 1 + 3 * NBUF]
        osem = scratch[1 + 3 * NBUF : 1 + 4 * NBUF]

        wid = lax.axis_index("s") * nc + lax.axis_index("c")
        chunk0 = wid * n_chunks
        pltpu.sync_copy(idx_hbm.at[pl.ds(chunk0, n_chunks)], idx_v)

        for b in range(NBUF):
            pltpu.async_copy(tbl_hbm.at[idx_v.at[b]], gbuf[b], gsem[b])

        def outer(i, carry):
            og = i * NBUF
            for b in range(NBUF):
                jg = og + b
                @pl.when(jg >= NBUF)
                def _():
                    pltpu.make_async_copy(
                        obuf[b],
                        out_hbm.at[pl.ds(0, out_rows_per_chunk)],
                        osem[b],
                    ).wait()

                pltpu.make_async_copy(
                    tbl_hbm.at[idx_v.at[jg]], gbuf[b], gsem[b]
                ).wait()

                def scale_pair(r2, _gb=gbuf[b], _ob=obuf[b]):
                    for h in range(2):
                        for c in range(D // LANES):
                            src = pl.ds(c * LANES, LANES)
                            dst = pl.ds(h * D + c * LANES, LANES)
                            _ob[r2, dst] = _gb[2 * r2 + h, src] * scale

                plsc.parallel_loop(0, out_rows_per_chunk, 1, unroll=2)(
                    scale_pair
                )

                @pl.when(jg + NBUF < n_chunks)
                def _():
                    pltpu.async_copy(
                        tbl_hbm.at[idx_v.at[jg + NBUF]], gbuf[b], gsem[b]
                    )

                pltpu.async_copy(
                    obuf[b],
                    out_hbm.at[
                        pl.ds((chunk0 + jg) * out_rows_per_chunk,
                              out_rows_per_chunk)
                    ],
                    osem[b],
                )
            return carry

        lax.fori_loop(0, n_chunks // NBUF, outer, 0)

        for b in range(NBUF):
            pltpu.make_async_copy(
                obuf[b], out_hbm.at[pl.ds(0, out_rows_per_chunk)], osem[b]
            ).wait()

    return emb


def kernel(x, table):
    b0, b1 = x.shape
    B = b0 * b1
    V, D = table.shape
    idx = x.reshape(B // CHUNK, CHUNK).astype(jnp.int32)
    t2 = lax.optimization_barrier(table.reshape(V * D // 128, 128))
    out = _build(B, V, D)(idx, t2.reshape(V, D))
    return out.reshape(b0, b1, D)


# speedup vs baseline: 1.4042x; 1.1111x over previous
import functools
import math

import jax
import jax.numpy as jnp
from jax import lax
from jax.experimental import pallas as pl
from jax.experimental.pallas import tpu as pltpu
from jax.experimental.pallas import tpu_sc as plsc

D_MODEL = 64
LANES = 16
NBUF = 4
SUB = 100


@functools.cache
def _build(B0: int, B1: int, V: int, D: int):
    info = plsc.get_sparse_core_info()
    nc, ns = info.num_cores, info.num_subcores
    nw = nc * ns
    rows_per_w = B0 // nw
    scale = math.sqrt(D)
    n_sub = 2 * rows_per_w
    orows = SUB * D // 1600

    mesh = plsc.VectorSubcoreMesh(core_axis_name="c", subcore_axis_name="s")

    @functools.partial(
        pl.kernel,
        out_type=jax.ShapeDtypeStruct(
            (B0, B1 * D // 1600, 1600), jnp.float32
        ),
        mesh=mesh,
        scratch_types=(
            [pltpu.VMEM((rows_per_w, SUB), jnp.int32) for _ in range(2)]
            + [pltpu.VMEM((SUB, D), jnp.float32) for _ in range(NBUF)]
            + [pltpu.VMEM((orows, 1600), jnp.float32) for _ in range(NBUF)]
            + [pltpu.SemaphoreType.DMA for _ in range(2 * NBUF)]
        ),
        compiler_params=pltpu.CompilerParams(use_tc_tiling_on_sc=False),
    )
    def emb(idx0_hbm, idx1_hbm, tbl_hbm, out_hbm, *scratch):
        idx_hbm = (idx0_hbm, idx1_hbm)
        idx_half = scratch[0:2]
        gbuf = scratch[2 : 2 + NBUF]
        obuf = scratch[2 + NBUF : 2 + 2 * NBUF]
        gsem = scratch[2 + 2 * NBUF : 2 + 3 * NBUF]
        osem = scratch[2 + 3 * NBUF : 2 + 4 * NBUF]

        wid = lax.axis_index("s") * nc + lax.axis_index("c")
        row0 = wid * rows_per_w
        for h in range(2):
            pltpu.sync_copy(
                idx_hbm[h].at[pl.ds(row0, rows_per_w)], idx_half[h]
            )

        def start_gather(r, h, b):
            pltpu.async_copy(
                tbl_hbm.at[idx_half[h].at[r]], gbuf[b], gsem[b]
            )

        def wait_gather(r, h, b):
            pltpu.make_async_copy(
                tbl_hbm.at[idx_half[h].at[r]], gbuf[b], gsem[b]
            ).wait()

        def wait_out(b):
            pltpu.make_async_copy(
                obuf[b], out_hbm.at[0, pl.ds(0, orows)], osem[b]
            ).wait()

        for s in range(NBUF):
            start_gather(s // 2, s % 2, s)

        def outer(i, carry):
            for q in range(NBUF):
                s = i * NBUF + q
                h = q % 2
                b = q
                r = 2 * i + q // 2

                @pl.when(s >= NBUF)
                def _():
                    wait_out(b)

                wait_gather(r, h, b)

                def scale_m(m, _gb=gbuf[b], _ob=obuf[b]):
                    mh = m // 4
                    ml = m % 4
                    src = pl.ds(ml * LANES, LANES)
                    for a in range(orows):
                        _ob[a, pl.ds(m * LANES, LANES)] = (
                            _gb[25 * a + mh, src] * scale
                        )

                plsc.parallel_loop(0, SUB, 1, unroll=2)(scale_m)

                @pl.when(s + NBUF < n_sub)
                def _():
                    start_gather(r + 2, h, b)

                pltpu.async_copy(
                    obuf[b],
                    out_hbm.at[row0 + r, pl.ds(h * orows, orows)],
                    osem[b],
                )
            return carry

        lax.fori_loop(0, n_sub // NBUF, outer, 0)

        for q in range(NBUF):
            wait_out(q)

    return emb


def kernel(x, table):
    b0, b1 = x.shape
    V, D = table.shape
    t2 = lax.optimization_barrier(table.reshape(V * D // 128, 128))
    xi = x.astype(jnp.int32)
    out = _build(b0, b1, V, D)(
        xi[:, :SUB], xi[:, SUB:], t2.reshape(V, D)
    )
    return out.reshape(b0, b1, D)


# speedup vs baseline: 1.4484x; 1.0315x over previous
import functools
import math

import jax
import jax.numpy as jnp
from jax import lax
from jax.experimental import pallas as pl
from jax.experimental.pallas import tpu as pltpu
from jax.experimental.pallas import tpu_sc as plsc

D_MODEL = 64
LANES = 16
SUB = 100


@functools.cache
def _build(B0: int, B1: int, V: int, D: int):
    info = plsc.get_sparse_core_info()
    nc, ns = info.num_cores, info.num_subcores
    nw = nc * ns
    rows_per_w = B0 // nw
    scale = math.sqrt(D)
    orows = B1 * D // 1600
    per_j = SUB * D // 1600

    mesh = plsc.VectorSubcoreMesh(core_axis_name="c", subcore_axis_name="s")

    @functools.partial(
        pl.kernel,
        out_type=jax.ShapeDtypeStruct((B0, orows, 1600), jnp.float32),
        mesh=mesh,
        scratch_types=(
            [pltpu.VMEM((rows_per_w, SUB), jnp.int32) for _ in range(4)]
            + [pltpu.VMEM((SUB, 128), jnp.float32) for _ in range(2)]
            + [pltpu.VMEM((orows, 1600), jnp.float32) for _ in range(2)]
            + [pltpu.SemaphoreType.DMA for _ in range(4)]
        ),
        compiler_params=pltpu.CompilerParams(
            use_tc_tiling_on_sc=True, needs_layout_passes=False
        ),
    )
    def emb(p0_hbm, p1_hbm, o0_hbm, o1_hbm, tbl_hbm, out_hbm, *scratch):
        pair_hbm = (p0_hbm, p1_hbm)
        off_hbm = (o0_hbm, o1_hbm)
        pair_v = scratch[0:2]
        off_v = scratch[2:4]
        gbuf = scratch[4:6]
        obuf = scratch[6:8]
        gsem = scratch[8:10]
        osem = scratch[10:12]

        wid = lax.axis_index("s") * nc + lax.axis_index("c")
        row0 = wid * rows_per_w
        for h in range(2):
            pltpu.sync_copy(
                pair_hbm[h].at[pl.ds(row0, rows_per_w)], pair_v[h]
            )
            pltpu.sync_copy(
                off_hbm[h].at[pl.ds(row0, rows_per_w)], off_v[h]
            )

        def start_gather(r, h):
            pltpu.async_copy(
                tbl_hbm.at[pair_v[h].at[r]], gbuf[h], gsem[h]
            )

        def wait_gather(r, h):
            pltpu.make_async_copy(
                tbl_hbm.at[pair_v[h].at[r]], gbuf[h], gsem[h]
            ).wait()

        def wait_out(ob):
            pltpu.make_async_copy(
                obuf[ob], out_hbm.at[0], osem[ob]
            ).wait()

        iota16 = lax.iota(jnp.int32, 16)

        for h in range(2):
            start_gather(0, h)

        def outer(i, carry):
            for rr in range(2):
                r = 2 * i + rr

                @pl.when(r >= 2)
                def _():
                    wait_out(rr)

                for h in range(2):
                    s = 2 * r + h
                    wait_gather(r, h)

                    def sel_j(j, _gb=gbuf[h], _ob=obuf[rr], _off=off_v[h],
                              _r=r, _h=h):
                        for a in range(per_j):
                            k = 25 * a + j
                            base = (k >> 4) << 4
                            lane = k & 15
                            offs = _off[_r, pl.ds(base, LANES)]
                            soff = lax.gather(
                                offs,
                                jnp.full((LANES, 1), lane, jnp.int32),
                                lax.GatherDimensionNumbers(
                                    offset_dims=(),
                                    collapsed_slice_dims=(0,),
                                    start_index_map=(0,),
                                ),
                                (1,),
                                mode=lax.GatherScatterMode.PROMISE_IN_BOUNDS,
                            )
                            row16 = jnp.full((LANES,), k, jnp.int32)
                            for c in range(D // LANES):
                                col16 = soff + (iota16 + c * LANES)
                                val = plsc.load_gather(
                                    _gb, [row16, col16]
                                )
                                _ob[
                                    per_j * _h + a,
                                    pl.ds(j * D + c * LANES, LANES),
                                ] = val * scale

                    plsc.parallel_loop(0, 1600 // D, 1, unroll=1)(sel_j)

                    @pl.when(r + 1 < rows_per_w)
                    def _():
                        start_gather(r + 1, h)

                pltpu.async_copy(
                    obuf[rr], out_hbm.at[row0 + r], osem[rr]
                )
            return carry

        lax.fori_loop(0, rows_per_w // 2, outer, 0)

        for ob in range(2):
            wait_out(ob)

    return emb


def kernel(x, table):
    b0, b1 = x.shape
    V, D = table.shape
    xi = x.astype(jnp.int32)
    xp = xi >> 1
    xo = (xi & 1) * D
    out = _build(b0, b1, V, D)(
        xp[:, :SUB],
        xp[:, SUB:],
        xo[:, :SUB],
        xo[:, SUB:],
        table.reshape(V * D // 128, 128),
    )
    return out.reshape(b0, b1, D)
